# Initial kernel scaffold; baseline (speedup 1.0000x reference)
import jax
import jax.numpy as jnp
from jax.experimental import pallas as pl


def kernel(memb, demb, pemb, mirna_edgelist, mirna_edgeweight, disease_edge_list, disease_edgeweight, pcg_edge_list, pcg_edgeweight, mirna_pcg_pairs, disease_pcg_pairs, mirna_disease_pairs, Wm1, bm1, Wm2, bm2, Wd1, bd1, Wd2, bd2, Wp1, bp1, Wp2, bp2, W_assoc, b_assoc, W_mp, b_mp, W_dp, b_dp):
    raise NotImplementedError("write your pallas kernel here")


# speedup vs baseline: 8.7419x; 8.7419x over previous
import functools

import jax
import jax.numpy as jnp
from jax import lax
from jax.experimental import pallas as pl
from jax.experimental.pallas import tpu as pltpu
from jax.experimental.pallas import tpu_sc as plsc

N = 10000
E = 320000
P = 100000
D = 128
H1 = 64
H2 = 32

NC = 2
NS = 16
NT = NC * NS
LANES = 16

EBLK = 8000
EBLK_DEG = 2000
PBLK = 4000
BN = 1000

_SC_PARAMS = pltpu.CompilerParams(needs_layout_passes=False)


@functools.cache
def _mesh():
    return plsc.VectorSubcoreMesh(
        core_axis_name="c", subcore_axis_name="s", num_cores=NC, num_subcores=NS
    )


def _wid():
    return lax.axis_index("s") * NC + lax.axis_index("c")


def _deg_body(el_hbm, w_hbm, out_hbm, ebuf, wbuf, acc):
    wid = _wid()
    zero16 = jnp.zeros((LANES,), jnp.float32)

    @pl.loop(0, N // LANES)
    def _(i):
        acc[pl.ds(i * LANES, LANES)] = zero16

    epw = E // NT
    base = wid * epw
    iota2 = lax.iota(jnp.int32, LANES) * 2 + 1

    @pl.loop(0, epw // EBLK_DEG)
    def _(b):
        off = base + b * EBLK_DEG
        pltpu.sync_copy(el_hbm.at[pl.ds(2 * off, 2 * EBLK_DEG)], ebuf)
        pltpu.sync_copy(w_hbm.at[pl.ds(off, EBLK_DEG)], wbuf)

        @pl.loop(0, EBLK_DEG // LANES)
        def _(c):
            cvec = plsc.load_gather(ebuf, [iota2 + c * (2 * LANES)])
            wv = wbuf[pl.ds(c * LANES, LANES)]
            plsc.addupdate_scatter(acc, [cvec], wv)

    pltpu.sync_copy(acc, out_hbm.at[pl.ds(wid * N, N)])


def _sc_deg(el, w):
    kern = pl.kernel(
        _deg_body,
        out_type=jax.ShapeDtypeStruct((NT * N,), jnp.float32),
        mesh=_mesh(),
        compiler_params=_SC_PARAMS,
        scratch_types=[
            pltpu.VMEM((2 * EBLK_DEG,), jnp.int32),
            pltpu.VMEM((EBLK_DEG,), jnp.float32),
            pltpu.VMEM((N,), jnp.float32),
        ],
    )
    return kern(el, w).reshape(NT, N)


def _scat_body(G, H, aT_hbm, el_hbm, w_hbm, out_hbm,
               ebuf, wbuf, t0, t1, t2, t3, a0, a1, a2, a3):
    wid = _wid()
    g = wid % G
    fg = wid // G
    fbase = fg * 4
    tables = (t0, t1, t2, t3)
    accs = (a0, a1, a2, a3)

    for j in range(4):
        pltpu.sync_copy(aT_hbm.at[pl.ds((fbase + j) * N, N)], tables[j])

    zero16 = jnp.zeros((LANES,), jnp.float32)

    @pl.loop(0, N // LANES)
    def _(i):
        sl = pl.ds(i * LANES, LANES)
        for j in range(4):
            accs[j][sl] = zero16

    epg = E // G
    base = g * epg
    iota2 = lax.iota(jnp.int32, LANES) * 2

    @pl.loop(0, epg // EBLK)
    def _(b):
        off = base + b * EBLK
        pltpu.sync_copy(el_hbm.at[pl.ds(2 * off, 2 * EBLK)], ebuf)
        pltpu.sync_copy(w_hbm.at[pl.ds(off, EBLK)], wbuf)

        @pl.loop(0, EBLK // LANES)
        def _(c):
            i2 = iota2 + c * (2 * LANES)
            rvec = plsc.load_gather(ebuf, [i2])
            cvec = plsc.load_gather(ebuf, [i2 + 1])
            wv = wbuf[pl.ds(c * LANES, LANES)]
            for j in range(4):
                gv = plsc.load_gather(tables[j], [rvec])
                plsc.addupdate_scatter(accs[j], [cvec], gv * wv)

    for j in range(4):
        pltpu.sync_copy(accs[j], out_hbm.at[pl.ds((g * H + fbase + j) * N, N)])


def _sc_scat(aT, el, w, G, H):
    kern = pl.kernel(
        functools.partial(_scat_body, G, H),
        out_type=jax.ShapeDtypeStruct((G * H * N,), jnp.float32),
        mesh=_mesh(),
        compiler_params=_SC_PARAMS,
        scratch_types=[
            pltpu.VMEM((2 * EBLK,), jnp.int32),
            pltpu.VMEM((EBLK,), jnp.float32),
        ] + [pltpu.VMEM((N,), jnp.float32) for _ in range(8)],
    )
    return kern(aT.reshape(H * N), el, w).reshape(G, H, N)


def _pair_body(pr_hbm, aT_hbm, bT_hbm, wv_hbm, out_hbm, pbuf, ta, tb, ob, wvb):
    wid = _wid()
    pltpu.sync_copy(aT_hbm.at[pl.ds(wid * N, N)], ta)
    pltpu.sync_copy(bT_hbm.at[pl.ds(wid * N, N)], tb)
    pltpu.sync_copy(wv_hbm, wvb)

    widv = jnp.full((LANES,), 0, jnp.int32) + wid
    wvec = plsc.load_gather(wvb, [widv])

    @pl.loop(0, N // LANES)
    def _(i):
        sl = pl.ds(i * LANES, LANES)
        ta[sl] = ta[sl] * wvec

    iota2 = lax.iota(jnp.int32, LANES) * 2

    @pl.loop(0, P // PBLK)
    def _(b):
        pltpu.sync_copy(pr_hbm.at[pl.ds(2 * b * PBLK, 2 * PBLK)], pbuf)

        @pl.loop(0, PBLK // LANES)
        def _(c):
            i2 = iota2 + c * (2 * LANES)
            ia = plsc.load_gather(pbuf, [i2])
            ib = plsc.load_gather(pbuf, [i2 + 1])
            ga = plsc.load_gather(ta, [ia])
            gb = plsc.load_gather(tb, [ib])
            ob[pl.ds(c * LANES, LANES)] = ga * gb

        pltpu.sync_copy(ob, out_hbm.at[pl.ds(wid * P + b * PBLK, PBLK)])


def _sc_pair(pairs, aT, bT, wcol):
    kern = pl.kernel(
        _pair_body,
        out_type=jax.ShapeDtypeStruct((NT * P,), jnp.float32),
        mesh=_mesh(),
        compiler_params=_SC_PARAMS,
        scratch_types=[
            pltpu.VMEM((2 * PBLK,), jnp.int32),
            pltpu.VMEM((N,), jnp.float32),
            pltpu.VMEM((N,), jnp.float32),
            pltpu.VMEM((PBLK,), jnp.float32),
            pltpu.VMEM((H2,), jnp.float32),
        ],
    )
    return kern(pairs, aT.reshape(NT * N), bT.reshape(NT * N), wcol).reshape(NT, P)


def _tcl1_body(x_ref, w_ref, b_ref, degp_ref, a_ref, s_ref, di_ref):
    deg = jnp.sum(degp_ref[...], axis=0) + 1.0
    dis = lax.rsqrt(deg)
    inv = 1.0 / deg
    xw = jnp.dot(x_ref[...], w_ref[...], preferred_element_type=jnp.float32)
    a_ref[...] = xw * dis[:, None]
    s_ref[...] = xw * inv[:, None] + b_ref[...]
    di_ref[...] = jnp.stack([dis, inv], axis=0)


def _tc_l1(x, W1, b1, degp):
    return pl.pallas_call(
        _tcl1_body,
        out_shape=[
            jax.ShapeDtypeStruct((N, H1), jnp.float32),
            jax.ShapeDtypeStruct((N, H1), jnp.float32),
            jax.ShapeDtypeStruct((2, N), jnp.float32),
        ],
    )(x, W1, b1, degp)


def _tcl2_body(acc_ref, s1_ref, di_ref, w_ref, b_ref, a_ref, s_ref):
    accsum = jnp.sum(acc_ref[...], axis=0)
    dis = di_ref[0]
    inv = di_ref[1]
    h1 = accsum * dis[:, None] + s1_ref[...]
    xw = jnp.dot(h1, w_ref[...], preferred_element_type=jnp.float32)
    a_ref[...] = xw * dis[:, None]
    s_ref[...] = xw * inv[:, None] + b_ref[...]


def _tc_l2(acc, s1, di, W2, b2):
    return pl.pallas_call(
        _tcl2_body,
        out_shape=[
            jax.ShapeDtypeStruct((N, H2), jnp.float32),
            jax.ShapeDtypeStruct((N, H2), jnp.float32),
        ],
    )(acc, s1, di, W2, b2)


def _tcfin_body(acc_ref, s2_ref, di_ref, o_ref):
    accsum = jnp.sum(acc_ref[...], axis=0)
    dis = di_ref[0]
    o_ref[...] = jax.nn.relu(accsum * dis[:, None] + s2_ref[...])


def _tc_fin(acc, s2, di):
    return pl.pallas_call(
        _tcfin_body,
        out_shape=jax.ShapeDtypeStruct((N, H2), jnp.float32),
    )(acc, s2, di)


def _tcsig_body(p_ref, b_ref, o_ref):
    s = jnp.sum(p_ref[...], axis=0) + b_ref[0, 0]
    o_ref[...] = jax.nn.sigmoid(s)[None, :]


def _tc_sig(partials, bias):
    out = pl.pallas_call(
        _tcsig_body,
        out_shape=jax.ShapeDtypeStruct((1, P), jnp.float32),
    )(partials, bias.reshape(1, 1))
    return out.reshape(P)


def _graph_embed(x, el, ew, W1, b1, W2, b2):
    el = el.reshape(2 * E)
    degp = _sc_deg(el, ew)
    A1, s1, di = _tc_l1(x, W1, b1.reshape(1, H1), degp)
    acc1 = _sc_scat(jnp.transpose(A1), el, ew, G=2, H=H1)
    A2, s2 = _tc_l2(jnp.transpose(acc1, (0, 2, 1)), s1, di, W2, b2.reshape(1, H2))
    acc2 = _sc_scat(jnp.transpose(A2), el, ew, G=4, H=H2)
    return _tc_fin(jnp.transpose(acc2, (0, 2, 1)), s2, di), di


def kernel(memb, demb, pemb, mirna_edgelist, mirna_edgeweight,
           disease_edge_list, disease_edgeweight, pcg_edge_list, pcg_edgeweight,
           mirna_pcg_pairs, disease_pcg_pairs, mirna_disease_pairs,
           Wm1, bm1, Wm2, bm2, Wd1, bd1, Wd2, bd2, Wp1, bp1, Wp2, bp2,
           W_assoc, b_assoc, W_mp, b_mp, W_dp, b_dp):
    mh, _ = _graph_embed(memb, mirna_edgelist, mirna_edgeweight, Wm1, bm1, Wm2, bm2)
    dh, _ = _graph_embed(demb, disease_edge_list, disease_edgeweight, Wd1, bd1, Wd2, bd2)
    ph, _ = _graph_embed(pemb, pcg_edge_list, pcg_edgeweight, Wp1, bp1, Wp2, bp2)

    mhT = jnp.transpose(mh)
    dhT = jnp.transpose(dh)
    phT = jnp.transpose(ph)

    pa = _sc_pair(mirna_disease_pairs.reshape(2 * P), mhT, dhT, W_assoc.reshape(H2))
    pm = _sc_pair(mirna_pcg_pairs.reshape(2 * P), mhT, phT, W_mp.reshape(H2))
    pd = _sc_pair(disease_pcg_pairs.reshape(2 * P), dhT, phT, W_dp.reshape(H2))

    assoc_out = _tc_sig(pa, b_assoc)
    mirna_pcg_out = _tc_sig(pm, b_mp)
    disease_pcg_out = _tc_sig(pd, b_dp)
    return (assoc_out, mirna_pcg_out, disease_pcg_out)


# speedup vs baseline: 14.0755x; 1.6101x over previous
import functools

import jax
import jax.numpy as jnp
from jax import lax
from jax.experimental import pallas as pl
from jax.experimental.pallas import tpu as pltpu
from jax.experimental.pallas import tpu_sc as plsc

N = 10000
E = 320000
P = 100000
D = 128
H1 = 64
H2 = 32

NC = 2
NS = 16
NT = NC * NS
LANES = 16

EBLK = 8000
EBLK_DEG = 2000
PBLK = 4000
BN = 1000

_SC_PARAMS = pltpu.CompilerParams(needs_layout_passes=False)


@functools.cache
def _mesh():
    return plsc.VectorSubcoreMesh(
        core_axis_name="c", subcore_axis_name="s", num_cores=NC, num_subcores=NS
    )


def _wid():
    return lax.axis_index("s") * NC + lax.axis_index("c")


def _deg_body(el_hbm, w_hbm, out_hbm, ebuf, wbuf, acc):
    wid = _wid()
    zero16 = jnp.zeros((LANES,), jnp.float32)

    @pl.loop(0, N // LANES)
    def _(i):
        acc[pl.ds(i * LANES, LANES)] = zero16

    epw = E // NT
    base = wid * epw
    iota2 = lax.iota(jnp.int32, LANES) * 2 + 1

    @pl.loop(0, epw // EBLK_DEG)
    def _(b):
        off = base + b * EBLK_DEG
        pltpu.sync_copy(el_hbm.at[pl.ds(2 * off, 2 * EBLK_DEG)], ebuf)
        pltpu.sync_copy(w_hbm.at[pl.ds(off, EBLK_DEG)], wbuf)

        @plsc.parallel_loop(0, EBLK_DEG // LANES, unroll=4)
        def _(c):
            cvec = plsc.load_gather(ebuf, [iota2 + c * (2 * LANES)])
            wv = wbuf[pl.ds(c * LANES, LANES)]
            plsc.addupdate_scatter(acc, [cvec], wv)

    pltpu.sync_copy(acc, out_hbm.at[pl.ds(wid * N, N)])


def _sc_deg(el, w):
    kern = pl.kernel(
        _deg_body,
        out_type=jax.ShapeDtypeStruct((NT * N,), jnp.float32),
        mesh=_mesh(),
        compiler_params=_SC_PARAMS,
        scratch_types=[
            pltpu.VMEM((2 * EBLK_DEG,), jnp.int32),
            pltpu.VMEM((EBLK_DEG,), jnp.float32),
            pltpu.VMEM((N,), jnp.float32),
        ],
    )
    return kern(el, w).reshape(NT, N)


def _scat_body(G, H, aT_hbm, el_hbm, w_hbm, out_hbm,
               ebuf, wbuf, t0, t1, t2, t3, a0, a1, a2, a3):
    wid = _wid()
    g = wid % G
    fg = wid // G
    fbase = fg * 4
    tables = (t0, t1, t2, t3)
    accs = (a0, a1, a2, a3)

    for j in range(4):
        pltpu.sync_copy(aT_hbm.at[pl.ds((fbase + j) * N, N)], tables[j])

    zero16 = jnp.zeros((LANES,), jnp.float32)

    @pl.loop(0, N // LANES)
    def _(i):
        sl = pl.ds(i * LANES, LANES)
        for j in range(4):
            accs[j][sl] = zero16

    epg = E // G
    base = g * epg
    iota2 = lax.iota(jnp.int32, LANES) * 2

    @pl.loop(0, epg // EBLK)
    def _(b):
        off = base + b * EBLK
        pltpu.sync_copy(el_hbm.at[pl.ds(2 * off, 2 * EBLK)], ebuf)
        pltpu.sync_copy(w_hbm.at[pl.ds(off, EBLK)], wbuf)

        @plsc.parallel_loop(0, EBLK // LANES, unroll=4)
        def _(c):
            i2 = iota2 + c * (2 * LANES)
            rvec = plsc.load_gather(ebuf, [i2])
            cvec = plsc.load_gather(ebuf, [i2 + 1])
            wv = wbuf[pl.ds(c * LANES, LANES)]
            for j in range(4):
                gv = plsc.load_gather(tables[j], [rvec])
                plsc.addupdate_scatter(accs[j], [cvec], gv * wv)

    for j in range(4):
        pltpu.sync_copy(accs[j], out_hbm.at[pl.ds((g * H + fbase + j) * N, N)])


def _sc_scat(aT, el, w, G, H):
    kern = pl.kernel(
        functools.partial(_scat_body, G, H),
        out_type=jax.ShapeDtypeStruct((G * H * N,), jnp.float32),
        mesh=_mesh(),
        compiler_params=_SC_PARAMS,
        scratch_types=[
            pltpu.VMEM((2 * EBLK,), jnp.int32),
            pltpu.VMEM((EBLK,), jnp.float32),
        ] + [pltpu.VMEM((N,), jnp.float32) for _ in range(8)],
    )
    return kern(aT.reshape(H * N), el, w).reshape(G, H, N)


def _pair_body(pr_hbm, aT_hbm, bT_hbm, wv_hbm, out_hbm, pbuf, ta, tb, ob, wvb):
    wid = _wid()
    pltpu.sync_copy(aT_hbm.at[pl.ds(wid * N, N)], ta)
    pltpu.sync_copy(bT_hbm.at[pl.ds(wid * N, N)], tb)
    pltpu.sync_copy(wv_hbm, wvb)

    widv = jnp.full((LANES,), 0, jnp.int32) + wid
    wvec = plsc.load_gather(wvb, [widv])

    @pl.loop(0, N // LANES)
    def _(i):
        sl = pl.ds(i * LANES, LANES)
        ta[sl] = ta[sl] * wvec

    iota2 = lax.iota(jnp.int32, LANES) * 2

    @pl.loop(0, P // PBLK)
    def _(b):
        pltpu.sync_copy(pr_hbm.at[pl.ds(2 * b * PBLK, 2 * PBLK)], pbuf)

        @plsc.parallel_loop(0, PBLK // LANES, unroll=4)
        def _(c):
            i2 = iota2 + c * (2 * LANES)
            ia = plsc.load_gather(pbuf, [i2])
            ib = plsc.load_gather(pbuf, [i2 + 1])
            ga = plsc.load_gather(ta, [ia])
            gb = plsc.load_gather(tb, [ib])
            ob[pl.ds(c * LANES, LANES)] = ga * gb

        pltpu.sync_copy(ob, out_hbm.at[pl.ds(wid * P + b * PBLK, PBLK)])


def _sc_pair(pairs, aT, bT, wcol):
    kern = pl.kernel(
        _pair_body,
        out_type=jax.ShapeDtypeStruct((NT * P,), jnp.float32),
        mesh=_mesh(),
        compiler_params=_SC_PARAMS,
        scratch_types=[
            pltpu.VMEM((2 * PBLK,), jnp.int32),
            pltpu.VMEM((N,), jnp.float32),
            pltpu.VMEM((N,), jnp.float32),
            pltpu.VMEM((PBLK,), jnp.float32),
            pltpu.VMEM((H2,), jnp.float32),
        ],
    )
    return kern(pairs, aT.reshape(NT * N), bT.reshape(NT * N), wcol).reshape(NT, P)


def _tcl1_body(x_ref, w_ref, b_ref, degp_ref, a_ref, s_ref, di_ref):
    deg = jnp.sum(degp_ref[...], axis=0) + 1.0
    dis = lax.rsqrt(deg)
    inv = 1.0 / deg
    xw = jnp.dot(x_ref[...], w_ref[...], preferred_element_type=jnp.float32)
    a_ref[...] = xw * dis[:, None]
    s_ref[...] = xw * inv[:, None] + b_ref[...]
    di_ref[...] = jnp.stack([dis, inv], axis=0)


def _tc_l1(x, W1, b1, degp):
    return pl.pallas_call(
        _tcl1_body,
        out_shape=[
            jax.ShapeDtypeStruct((N, H1), jnp.float32),
            jax.ShapeDtypeStruct((N, H1), jnp.float32),
            jax.ShapeDtypeStruct((2, N), jnp.float32),
        ],
    )(x, W1, b1, degp)


def _tcl2_body(acc_ref, s1_ref, di_ref, w_ref, b_ref, a_ref, s_ref):
    accsum = jnp.sum(acc_ref[...], axis=0)
    dis = di_ref[0]
    inv = di_ref[1]
    h1 = accsum * dis[:, None] + s1_ref[...]
    xw = jnp.dot(h1, w_ref[...], preferred_element_type=jnp.float32)
    a_ref[...] = xw * dis[:, None]
    s_ref[...] = xw * inv[:, None] + b_ref[...]


def _tc_l2(acc, s1, di, W2, b2):
    return pl.pallas_call(
        _tcl2_body,
        out_shape=[
            jax.ShapeDtypeStruct((N, H2), jnp.float32),
            jax.ShapeDtypeStruct((N, H2), jnp.float32),
        ],
    )(acc, s1, di, W2, b2)


def _tcfin_body(acc_ref, s2_ref, di_ref, o_ref):
    accsum = jnp.sum(acc_ref[...], axis=0)
    dis = di_ref[0]
    o_ref[...] = jax.nn.relu(accsum * dis[:, None] + s2_ref[...])


def _tc_fin(acc, s2, di):
    return pl.pallas_call(
        _tcfin_body,
        out_shape=jax.ShapeDtypeStruct((N, H2), jnp.float32),
    )(acc, s2, di)


def _tcsig_body(p_ref, b_ref, o_ref):
    s = jnp.sum(p_ref[...], axis=0) + b_ref[0, 0]
    o_ref[...] = jax.nn.sigmoid(s)[None, :]


def _tc_sig(partials, bias):
    out = pl.pallas_call(
        _tcsig_body,
        out_shape=jax.ShapeDtypeStruct((1, P), jnp.float32),
    )(partials, bias.reshape(1, 1))
    return out.reshape(P)


def _graph_embed(x, el, ew, W1, b1, W2, b2):
    el = el.reshape(2 * E)
    degp = _sc_deg(el, ew)
    A1, s1, di = _tc_l1(x, W1, b1.reshape(1, H1), degp)
    acc1 = _sc_scat(jnp.transpose(A1), el, ew, G=2, H=H1)
    A2, s2 = _tc_l2(jnp.transpose(acc1, (0, 2, 1)), s1, di, W2, b2.reshape(1, H2))
    acc2 = _sc_scat(jnp.transpose(A2), el, ew, G=4, H=H2)
    return _tc_fin(jnp.transpose(acc2, (0, 2, 1)), s2, di), di


def kernel(memb, demb, pemb, mirna_edgelist, mirna_edgeweight,
           disease_edge_list, disease_edgeweight, pcg_edge_list, pcg_edgeweight,
           mirna_pcg_pairs, disease_pcg_pairs, mirna_disease_pairs,
           Wm1, bm1, Wm2, bm2, Wd1, bd1, Wd2, bd2, Wp1, bp1, Wp2, bp2,
           W_assoc, b_assoc, W_mp, b_mp, W_dp, b_dp):
    mh, _ = _graph_embed(memb, mirna_edgelist, mirna_edgeweight, Wm1, bm1, Wm2, bm2)
    dh, _ = _graph_embed(demb, disease_edge_list, disease_edgeweight, Wd1, bd1, Wd2, bd2)
    ph, _ = _graph_embed(pemb, pcg_edge_list, pcg_edgeweight, Wp1, bp1, Wp2, bp2)

    mhT = jnp.transpose(mh)
    dhT = jnp.transpose(dh)
    phT = jnp.transpose(ph)

    pa = _sc_pair(mirna_disease_pairs.reshape(2 * P), mhT, dhT, W_assoc.reshape(H2))
    pm = _sc_pair(mirna_pcg_pairs.reshape(2 * P), mhT, phT, W_mp.reshape(H2))
    pd = _sc_pair(disease_pcg_pairs.reshape(2 * P), dhT, phT, W_dp.reshape(H2))

    assoc_out = _tc_sig(pa, b_assoc)
    mirna_pcg_out = _tc_sig(pm, b_mp)
    disease_pcg_out = _tc_sig(pd, b_dp)
    return (assoc_out, mirna_pcg_out, disease_pcg_out)


# speedup vs baseline: 16.1968x; 1.1507x over previous
import functools

import jax
import jax.numpy as jnp
from jax import lax
from jax.experimental import pallas as pl
from jax.experimental.pallas import tpu as pltpu
from jax.experimental.pallas import tpu_sc as plsc

N = 10000
E = 320000
P = 100000
D = 128
H1 = 64
H2 = 32

NC = 2
NS = 16
NT = NC * NS
LANES = 16

EBLK = 4000
EBLK_DEG = 2000
PBLK = 2000
BN = 1000

_SC_PARAMS = pltpu.CompilerParams(needs_layout_passes=False)


@functools.cache
def _mesh():
    return plsc.VectorSubcoreMesh(
        core_axis_name="c", subcore_axis_name="s", num_cores=NC, num_subcores=NS
    )


def _wid():
    return lax.axis_index("s") * NC + lax.axis_index("c")


def _deg_body(el_hbm, w_hbm, out_hbm, ebuf, wbuf, acc):
    wid = _wid()
    zero16 = jnp.zeros((LANES,), jnp.float32)

    @pl.loop(0, N // LANES)
    def _(i):
        acc[pl.ds(i * LANES, LANES)] = zero16

    epw = E // NT
    base = wid * epw
    iota2 = lax.iota(jnp.int32, LANES) * 2 + 1

    @pl.loop(0, epw // EBLK_DEG)
    def _(b):
        off = base + b * EBLK_DEG
        pltpu.sync_copy(el_hbm.at[pl.ds(2 * off, 2 * EBLK_DEG)], ebuf)
        pltpu.sync_copy(w_hbm.at[pl.ds(off, EBLK_DEG)], wbuf)

        @plsc.parallel_loop(0, EBLK_DEG // LANES, unroll=4)
        def _(c):
            cvec = plsc.load_gather(ebuf, [iota2 + c * (2 * LANES)])
            wv = wbuf[pl.ds(c * LANES, LANES)]
            plsc.addupdate_scatter(acc, [cvec], wv)

    pltpu.sync_copy(acc, out_hbm.at[pl.ds(wid * N, N)])


def _sc_deg(el, w):
    kern = pl.kernel(
        _deg_body,
        out_type=jax.ShapeDtypeStruct((NT * N,), jnp.float32),
        mesh=_mesh(),
        compiler_params=_SC_PARAMS,
        scratch_types=[
            pltpu.VMEM((2 * EBLK_DEG,), jnp.int32),
            pltpu.VMEM((EBLK_DEG,), jnp.float32),
            pltpu.VMEM((N,), jnp.float32),
        ],
    )
    return kern(el, w).reshape(NT, N)


def _scat_body(G, H, aT_hbm, el_hbm, w_hbm, out_hbm,
               ebuf0, wbuf0, ebuf1, wbuf1, t0, t1, t2, t3, a0, a1, a2, a3,
               se0, sw0, se1, sw1):
    wid = _wid()
    g = wid % G
    fg = wid // G
    fbase = fg * 4
    tables = (t0, t1, t2, t3)
    accs = (a0, a1, a2, a3)

    for j in range(4):
        pltpu.sync_copy(aT_hbm.at[pl.ds((fbase + j) * N, N)], tables[j])

    zero16 = jnp.zeros((LANES,), jnp.float32)

    @pl.loop(0, N // LANES)
    def _(i):
        sl = pl.ds(i * LANES, LANES)
        for j in range(4):
            accs[j][sl] = zero16

    epg = E // G
    base = g * epg
    nblk = epg // EBLK
    iota2 = lax.iota(jnp.int32, LANES) * 2

    def start(b, ebuf, wbuf, se, sw):
        off = base + b * EBLK
        pltpu.async_copy(el_hbm.at[pl.ds(2 * off, 2 * EBLK)], ebuf, se)
        pltpu.async_copy(w_hbm.at[pl.ds(off, EBLK)], wbuf, sw)

    def wait(ebuf, wbuf, se, sw):
        pltpu.make_async_copy(el_hbm.at[pl.ds(0, 2 * EBLK)], ebuf, se).wait()
        pltpu.make_async_copy(w_hbm.at[pl.ds(0, EBLK)], wbuf, sw).wait()

    def compute(ebuf, wbuf):
        @plsc.parallel_loop(0, EBLK // LANES, unroll=8)
        def _(c):
            i2 = iota2 + c * (2 * LANES)
            rvec = plsc.load_gather(ebuf, [i2])
            cvec = plsc.load_gather(ebuf, [i2 + 1])
            wv = wbuf[pl.ds(c * LANES, LANES)]
            for j in range(4):
                gv = plsc.load_gather(tables[j], [rvec])
                plsc.addupdate_scatter(accs[j], [cvec], gv * wv)

    start(0, ebuf0, wbuf0, se0, sw0)

    @pl.loop(0, nblk // 2)
    def _(h):
        b0 = 2 * h
        start(b0 + 1, ebuf1, wbuf1, se1, sw1)
        wait(ebuf0, wbuf0, se0, sw0)
        compute(ebuf0, wbuf0)

        @pl.when(b0 + 2 < nblk)
        def _():
            start(b0 + 2, ebuf0, wbuf0, se0, sw0)

        wait(ebuf1, wbuf1, se1, sw1)
        compute(ebuf1, wbuf1)

    for j in range(4):
        pltpu.sync_copy(accs[j], out_hbm.at[pl.ds((g * H + fbase + j) * N, N)])


def _sc_scat(aT, el, w, G, H):
    kern = pl.kernel(
        functools.partial(_scat_body, G, H),
        out_type=jax.ShapeDtypeStruct((G * H * N,), jnp.float32),
        mesh=_mesh(),
        compiler_params=_SC_PARAMS,
        scratch_types=[
            pltpu.VMEM((2 * EBLK,), jnp.int32),
            pltpu.VMEM((EBLK,), jnp.float32),
            pltpu.VMEM((2 * EBLK,), jnp.int32),
            pltpu.VMEM((EBLK,), jnp.float32),
        ] + [pltpu.VMEM((N,), jnp.float32) for _ in range(8)]
          + [pltpu.SemaphoreType.DMA for _ in range(4)],
    )
    return kern(aT.reshape(H * N), el, w).reshape(G, H, N)


def _pair_body(pr_hbm, aT_hbm, bT_hbm, wv_hbm, out_hbm,
               pbuf0, pbuf1, ta, tb, ob0, ob1, wvb,
               si0, si1, so0, so1):
    wid = _wid()
    pltpu.sync_copy(aT_hbm.at[pl.ds(wid * N, N)], ta)
    pltpu.sync_copy(bT_hbm.at[pl.ds(wid * N, N)], tb)
    pltpu.sync_copy(wv_hbm, wvb)

    widv = jnp.full((LANES,), 0, jnp.int32) + wid
    wvec = plsc.load_gather(wvb, [widv])

    @pl.loop(0, N // LANES)
    def _(i):
        sl = pl.ds(i * LANES, LANES)
        ta[sl] = ta[sl] * wvec

    iota2 = lax.iota(jnp.int32, LANES) * 2
    nblk = P // PBLK

    def start_in(b, pbuf, si):
        pltpu.async_copy(pr_hbm.at[pl.ds(2 * b * PBLK, 2 * PBLK)], pbuf, si)

    def wait_in(pbuf, si):
        pltpu.make_async_copy(pr_hbm.at[pl.ds(0, 2 * PBLK)], pbuf, si).wait()

    def compute(pbuf, ob):
        @plsc.parallel_loop(0, PBLK // LANES, unroll=8)
        def _(c):
            i2 = iota2 + c * (2 * LANES)
            ia = plsc.load_gather(pbuf, [i2])
            ib = plsc.load_gather(pbuf, [i2 + 1])
            ga = plsc.load_gather(ta, [ia])
            gb = plsc.load_gather(tb, [ib])
            ob[pl.ds(c * LANES, LANES)] = ga * gb

    def start_out(b, ob, so):
        pltpu.async_copy(ob, out_hbm.at[pl.ds(wid * P + b * PBLK, PBLK)], so)

    def wait_out(ob, so):
        pltpu.make_async_copy(ob, out_hbm.at[pl.ds(0, PBLK)], so).wait()

    start_in(0, pbuf0, si0)

    @pl.loop(0, nblk // 2)
    def _(h):
        b0 = 2 * h
        start_in(b0 + 1, pbuf1, si1)
        wait_in(pbuf0, si0)

        @pl.when(h > 0)
        def _():
            wait_out(ob0, so0)

        compute(pbuf0, ob0)
        start_out(b0, ob0, so0)

        @pl.when(b0 + 2 < nblk)
        def _():
            start_in(b0 + 2, pbuf0, si0)

        wait_in(pbuf1, si1)

        @pl.when(h > 0)
        def _():
            wait_out(ob1, so1)

        compute(pbuf1, ob1)
        start_out(b0 + 1, ob1, so1)

    wait_out(ob0, so0)
    wait_out(ob1, so1)


def _sc_pair(pairs, aT, bT, wcol):
    kern = pl.kernel(
        _pair_body,
        out_type=jax.ShapeDtypeStruct((NT * P,), jnp.float32),
        mesh=_mesh(),
        compiler_params=_SC_PARAMS,
        scratch_types=[
            pltpu.VMEM((2 * PBLK,), jnp.int32),
            pltpu.VMEM((2 * PBLK,), jnp.int32),
            pltpu.VMEM((N,), jnp.float32),
            pltpu.VMEM((N,), jnp.float32),
            pltpu.VMEM((PBLK,), jnp.float32),
            pltpu.VMEM((PBLK,), jnp.float32),
            pltpu.VMEM((H2,), jnp.float32),
        ] + [pltpu.SemaphoreType.DMA for _ in range(4)],
    )
    return kern(pairs, aT.reshape(NT * N), bT.reshape(NT * N), wcol).reshape(NT, P)


def _tcl1_body(x_ref, w_ref, b_ref, degp_ref, a_ref, s_ref, di_ref):
    deg = jnp.sum(degp_ref[...], axis=0) + 1.0
    dis = lax.rsqrt(deg)
    inv = 1.0 / deg
    xw = jnp.dot(x_ref[...], w_ref[...], preferred_element_type=jnp.float32)
    a_ref[...] = xw * dis[:, None]
    s_ref[...] = xw * inv[:, None] + b_ref[...]
    di_ref[...] = jnp.stack([dis, inv], axis=0)


def _tc_l1(x, W1, b1, degp):
    return pl.pallas_call(
        _tcl1_body,
        out_shape=[
            jax.ShapeDtypeStruct((N, H1), jnp.float32),
            jax.ShapeDtypeStruct((N, H1), jnp.float32),
            jax.ShapeDtypeStruct((2, N), jnp.float32),
        ],
    )(x, W1, b1, degp)


def _tcl2_body(acc_ref, s1_ref, di_ref, w_ref, b_ref, a_ref, s_ref):
    accsum = jnp.sum(acc_ref[...], axis=0)
    dis = di_ref[0]
    inv = di_ref[1]
    h1 = accsum * dis[:, None] + s1_ref[...]
    xw = jnp.dot(h1, w_ref[...], preferred_element_type=jnp.float32)
    a_ref[...] = xw * dis[:, None]
    s_ref[...] = xw * inv[:, None] + b_ref[...]


def _tc_l2(acc, s1, di, W2, b2):
    return pl.pallas_call(
        _tcl2_body,
        out_shape=[
            jax.ShapeDtypeStruct((N, H2), jnp.float32),
            jax.ShapeDtypeStruct((N, H2), jnp.float32),
        ],
    )(acc, s1, di, W2, b2)


def _tcfin_body(acc_ref, s2_ref, di_ref, o_ref):
    accsum = jnp.sum(acc_ref[...], axis=0)
    dis = di_ref[0]
    o_ref[...] = jax.nn.relu(accsum * dis[:, None] + s2_ref[...])


def _tc_fin(acc, s2, di):
    return pl.pallas_call(
        _tcfin_body,
        out_shape=jax.ShapeDtypeStruct((N, H2), jnp.float32),
    )(acc, s2, di)


def _tcsig_body(p_ref, b_ref, o_ref):
    s = jnp.sum(p_ref[...], axis=0) + b_ref[0, 0]
    o_ref[...] = jax.nn.sigmoid(s)[None, :]


def _tc_sig(partials, bias):
    out = pl.pallas_call(
        _tcsig_body,
        out_shape=jax.ShapeDtypeStruct((1, P), jnp.float32),
    )(partials, bias.reshape(1, 1))
    return out.reshape(P)


def _graph_embed(x, el, ew, W1, b1, W2, b2):
    el = el.reshape(2 * E)
    degp = _sc_deg(el, ew)
    A1, s1, di = _tc_l1(x, W1, b1.reshape(1, H1), degp)
    acc1 = _sc_scat(jnp.transpose(A1), el, ew, G=2, H=H1)
    A2, s2 = _tc_l2(jnp.transpose(acc1, (0, 2, 1)), s1, di, W2, b2.reshape(1, H2))
    acc2 = _sc_scat(jnp.transpose(A2), el, ew, G=4, H=H2)
    return _tc_fin(jnp.transpose(acc2, (0, 2, 1)), s2, di), di


def kernel(memb, demb, pemb, mirna_edgelist, mirna_edgeweight,
           disease_edge_list, disease_edgeweight, pcg_edge_list, pcg_edgeweight,
           mirna_pcg_pairs, disease_pcg_pairs, mirna_disease_pairs,
           Wm1, bm1, Wm2, bm2, Wd1, bd1, Wd2, bd2, Wp1, bp1, Wp2, bp2,
           W_assoc, b_assoc, W_mp, b_mp, W_dp, b_dp):
    mh, _ = _graph_embed(memb, mirna_edgelist, mirna_edgeweight, Wm1, bm1, Wm2, bm2)
    dh, _ = _graph_embed(demb, disease_edge_list, disease_edgeweight, Wd1, bd1, Wd2, bd2)
    ph, _ = _graph_embed(pemb, pcg_edge_list, pcg_edgeweight, Wp1, bp1, Wp2, bp2)

    mhT = jnp.transpose(mh)
    dhT = jnp.transpose(dh)
    phT = jnp.transpose(ph)

    pa = _sc_pair(mirna_disease_pairs.reshape(2 * P), mhT, dhT, W_assoc.reshape(H2))
    pm = _sc_pair(mirna_pcg_pairs.reshape(2 * P), mhT, phT, W_mp.reshape(H2))
    pd = _sc_pair(disease_pcg_pairs.reshape(2 * P), dhT, phT, W_dp.reshape(H2))

    assoc_out = _tc_sig(pa, b_assoc)
    mirna_pcg_out = _tc_sig(pm, b_mp)
    disease_pcg_out = _tc_sig(pd, b_dp)
    return (assoc_out, mirna_pcg_out, disease_pcg_out)


# speedup vs baseline: 26.2578x; 1.6212x over previous
import functools

import jax
import jax.numpy as jnp
from jax import lax
from jax.experimental import pallas as pl
from jax.experimental.pallas import tpu as pltpu
from jax.experimental.pallas import tpu_sc as plsc

N = 10000
E = 320000
P = 100000
D = 128
H1 = 64
H2 = 32

NC = 2
NS = 16
NT = NC * NS
LANES = 16

EBLK = 4000
EBLK_DEG = 2000
PBLK = 2000
BN = 1000

_SC_PARAMS = pltpu.CompilerParams(needs_layout_passes=False)


@functools.cache
def _mesh():
    return plsc.VectorSubcoreMesh(
        core_axis_name="c", subcore_axis_name="s", num_cores=NC, num_subcores=NS
    )


def _wid():
    return lax.axis_index("s") * NC + lax.axis_index("c")


def _deg_body(el_hbm, w_hbm, out_hbm, cbuf, wbuf, acc):
    wid = _wid()
    zero16 = jnp.zeros((LANES,), jnp.float32)

    @pl.loop(0, N // LANES)
    def _(i):
        acc[pl.ds(i * LANES, LANES)] = zero16

    epw = E // NT
    base = wid * epw

    @pl.loop(0, epw // EBLK_DEG)
    def _(b):
        off = base + b * EBLK_DEG
        pltpu.sync_copy(el_hbm.at[pl.ds(E + off, EBLK_DEG)], cbuf)
        pltpu.sync_copy(w_hbm.at[pl.ds(off, EBLK_DEG)], wbuf)

        @plsc.parallel_loop(0, EBLK_DEG // LANES, unroll=4)
        def _(c):
            sl = pl.ds(c * LANES, LANES)
            plsc.addupdate_scatter(acc, [cbuf[sl]], wbuf[sl])

    pltpu.sync_copy(acc, out_hbm.at[pl.ds(wid * N, N)])


def _sc_deg(el, w):
    kern = pl.kernel(
        _deg_body,
        out_type=jax.ShapeDtypeStruct((NT * N,), jnp.float32),
        mesh=_mesh(),
        compiler_params=_SC_PARAMS,
        scratch_types=[
            pltpu.VMEM((EBLK_DEG,), jnp.int32),
            pltpu.VMEM((EBLK_DEG,), jnp.float32),
            pltpu.VMEM((N,), jnp.float32),
        ],
    )
    return kern(el, w).reshape(NT, N)


def _scat_body(G, H, aT_hbm, el_hbm, w_hbm, out_hbm,
               rbuf0, cbuf0, wbuf0, rbuf1, cbuf1, wbuf1,
               t0, t1, t2, t3, a0, a1, a2, a3,
               s0, s1):
    wid = _wid()
    g = wid % G
    fg = wid // G
    fbase = fg * 4
    tables = (t0, t1, t2, t3)
    accs = (a0, a1, a2, a3)

    for j in range(4):
        pltpu.sync_copy(aT_hbm.at[pl.ds((fbase + j) * N, N)], tables[j])

    zero16 = jnp.zeros((LANES,), jnp.float32)

    @pl.loop(0, N // LANES)
    def _(i):
        sl = pl.ds(i * LANES, LANES)
        for j in range(4):
            accs[j][sl] = zero16

    epg = E // G
    base = g * epg
    nblk = epg // EBLK

    def start(b, rbuf, cbuf, wbuf, sem):
        off = base + b * EBLK
        pltpu.async_copy(el_hbm.at[pl.ds(off, EBLK)], rbuf, sem)
        pltpu.async_copy(el_hbm.at[pl.ds(E + off, EBLK)], cbuf, sem)
        pltpu.async_copy(w_hbm.at[pl.ds(off, EBLK)], wbuf, sem)

    def wait(rbuf, cbuf, wbuf, sem):
        pltpu.make_async_copy(el_hbm.at[pl.ds(0, EBLK)], rbuf, sem).wait()
        pltpu.make_async_copy(el_hbm.at[pl.ds(0, EBLK)], cbuf, sem).wait()
        pltpu.make_async_copy(w_hbm.at[pl.ds(0, EBLK)], wbuf, sem).wait()

    def compute(rbuf, cbuf, wbuf):
        @plsc.parallel_loop(0, EBLK // LANES, unroll=8)
        def _(c):
            sl = pl.ds(c * LANES, LANES)
            rvec = rbuf[sl]
            cvec = cbuf[sl]
            wv = wbuf[sl]
            for j in range(4):
                gv = plsc.load_gather(tables[j], [rvec])
                plsc.addupdate_scatter(accs[j], [cvec], gv * wv)

    start(0, rbuf0, cbuf0, wbuf0, s0)

    @pl.loop(0, nblk // 2)
    def _(h):
        b0 = 2 * h
        start(b0 + 1, rbuf1, cbuf1, wbuf1, s1)
        wait(rbuf0, cbuf0, wbuf0, s0)
        compute(rbuf0, cbuf0, wbuf0)

        @pl.when(b0 + 2 < nblk)
        def _():
            start(b0 + 2, rbuf0, cbuf0, wbuf0, s0)

        wait(rbuf1, cbuf1, wbuf1, s1)
        compute(rbuf1, cbuf1, wbuf1)

    for j in range(4):
        pltpu.sync_copy(accs[j], out_hbm.at[pl.ds((g * H + fbase + j) * N, N)])


def _sc_scat(aT, el, w, G, H):
    kern = pl.kernel(
        functools.partial(_scat_body, G, H),
        out_type=jax.ShapeDtypeStruct((G * H * N,), jnp.float32),
        mesh=_mesh(),
        compiler_params=_SC_PARAMS,
        scratch_types=[
            pltpu.VMEM((EBLK,), jnp.int32),
            pltpu.VMEM((EBLK,), jnp.int32),
            pltpu.VMEM((EBLK,), jnp.float32),
            pltpu.VMEM((EBLK,), jnp.int32),
            pltpu.VMEM((EBLK,), jnp.int32),
            pltpu.VMEM((EBLK,), jnp.float32),
        ] + [pltpu.VMEM((N,), jnp.float32) for _ in range(8)]
          + [pltpu.SemaphoreType.DMA, pltpu.SemaphoreType.DMA],
    )
    return kern(aT.reshape(H * N), el, w).reshape(G, H, N)


def _pair_body(pr_hbm, aT_hbm, bT_hbm, wv_hbm, out_hbm,
               ibuf0, jbuf0, ibuf1, jbuf1, ta, tb, ob0, ob1, wvb,
               si0, si1, so0, so1):
    wid = _wid()
    pltpu.sync_copy(aT_hbm.at[pl.ds(wid * N, N)], ta)
    pltpu.sync_copy(bT_hbm.at[pl.ds(wid * N, N)], tb)
    pltpu.sync_copy(wv_hbm, wvb)

    widv = jnp.full((LANES,), 0, jnp.int32) + wid
    wvec = plsc.load_gather(wvb, [widv])

    @pl.loop(0, N // LANES)
    def _(i):
        sl = pl.ds(i * LANES, LANES)
        ta[sl] = ta[sl] * wvec

    nblk = P // PBLK

    def start_in(b, ibuf, jbuf, si):
        pltpu.async_copy(pr_hbm.at[pl.ds(b * PBLK, PBLK)], ibuf, si)
        pltpu.async_copy(pr_hbm.at[pl.ds(P + b * PBLK, PBLK)], jbuf, si)

    def wait_in(ibuf, jbuf, si):
        pltpu.make_async_copy(pr_hbm.at[pl.ds(0, PBLK)], ibuf, si).wait()
        pltpu.make_async_copy(pr_hbm.at[pl.ds(0, PBLK)], jbuf, si).wait()

    def compute(ibuf, jbuf, ob):
        @plsc.parallel_loop(0, PBLK // LANES, unroll=8)
        def _(c):
            sl = pl.ds(c * LANES, LANES)
            ga = plsc.load_gather(ta, [ibuf[sl]])
            gb = plsc.load_gather(tb, [jbuf[sl]])
            ob[sl] = ga * gb

    def start_out(b, ob, so):
        pltpu.async_copy(ob, out_hbm.at[pl.ds(wid * P + b * PBLK, PBLK)], so)

    def wait_out(ob, so):
        pltpu.make_async_copy(ob, out_hbm.at[pl.ds(0, PBLK)], so).wait()

    start_in(0, ibuf0, jbuf0, si0)

    @pl.loop(0, nblk // 2)
    def _(h):
        b0 = 2 * h
        start_in(b0 + 1, ibuf1, jbuf1, si1)
        wait_in(ibuf0, jbuf0, si0)

        @pl.when(h > 0)
        def _():
            wait_out(ob0, so0)

        compute(ibuf0, jbuf0, ob0)
        start_out(b0, ob0, so0)

        @pl.when(b0 + 2 < nblk)
        def _():
            start_in(b0 + 2, ibuf0, jbuf0, si0)

        wait_in(ibuf1, jbuf1, si1)

        @pl.when(h > 0)
        def _():
            wait_out(ob1, so1)

        compute(ibuf1, jbuf1, ob1)
        start_out(b0 + 1, ob1, so1)

    wait_out(ob0, so0)
    wait_out(ob1, so1)


def _sc_pair(pairs, aT, bT, wcol):
    kern = pl.kernel(
        _pair_body,
        out_type=jax.ShapeDtypeStruct((NT * P,), jnp.float32),
        mesh=_mesh(),
        compiler_params=_SC_PARAMS,
        scratch_types=[
            pltpu.VMEM((PBLK,), jnp.int32),
            pltpu.VMEM((PBLK,), jnp.int32),
            pltpu.VMEM((PBLK,), jnp.int32),
            pltpu.VMEM((PBLK,), jnp.int32),
            pltpu.VMEM((N,), jnp.float32),
            pltpu.VMEM((N,), jnp.float32),
            pltpu.VMEM((PBLK,), jnp.float32),
            pltpu.VMEM((PBLK,), jnp.float32),
            pltpu.VMEM((H2,), jnp.float32),
        ] + [pltpu.SemaphoreType.DMA for _ in range(4)],
    )
    return kern(pairs, aT.reshape(NT * N), bT.reshape(NT * N), wcol).reshape(NT, P)


def _tcl1_body(x_ref, w_ref, b_ref, degp_ref, a_ref, s_ref, di_ref):
    deg = jnp.sum(degp_ref[...], axis=0) + 1.0
    dis = lax.rsqrt(deg)
    inv = 1.0 / deg
    xw = jnp.dot(x_ref[...], w_ref[...], preferred_element_type=jnp.float32)
    a_ref[...] = xw * dis[:, None]
    s_ref[...] = xw * inv[:, None] + b_ref[...]
    di_ref[...] = jnp.stack([dis, inv], axis=0)


def _tc_l1(x, W1, b1, degp):
    return pl.pallas_call(
        _tcl1_body,
        out_shape=[
            jax.ShapeDtypeStruct((N, H1), jnp.float32),
            jax.ShapeDtypeStruct((N, H1), jnp.float32),
            jax.ShapeDtypeStruct((2, N), jnp.float32),
        ],
    )(x, W1, b1, degp)


def _tcl2_body(acc_ref, s1_ref, di_ref, w_ref, b_ref, a_ref, s_ref):
    accsum = jnp.sum(acc_ref[...], axis=0)
    dis = di_ref[0]
    inv = di_ref[1]
    h1 = accsum * dis[:, None] + s1_ref[...]
    xw = jnp.dot(h1, w_ref[...], preferred_element_type=jnp.float32)
    a_ref[...] = xw * dis[:, None]
    s_ref[...] = xw * inv[:, None] + b_ref[...]


def _tc_l2(acc, s1, di, W2, b2):
    return pl.pallas_call(
        _tcl2_body,
        out_shape=[
            jax.ShapeDtypeStruct((N, H2), jnp.float32),
            jax.ShapeDtypeStruct((N, H2), jnp.float32),
        ],
    )(acc, s1, di, W2, b2)


def _tcfin_body(acc_ref, s2_ref, di_ref, o_ref):
    accsum = jnp.sum(acc_ref[...], axis=0)
    dis = di_ref[0]
    o_ref[...] = jax.nn.relu(accsum * dis[:, None] + s2_ref[...])


def _tc_fin(acc, s2, di):
    return pl.pallas_call(
        _tcfin_body,
        out_shape=jax.ShapeDtypeStruct((N, H2), jnp.float32),
    )(acc, s2, di)


def _tcsig_body(p_ref, b_ref, o_ref):
    s = jnp.sum(p_ref[...], axis=0) + b_ref[0, 0]
    o_ref[...] = jax.nn.sigmoid(s)[None, :]


def _tc_sig(partials, bias):
    out = pl.pallas_call(
        _tcsig_body,
        out_shape=jax.ShapeDtypeStruct((1, P), jnp.float32),
    )(partials, bias.reshape(1, 1))
    return out.reshape(P)


def _graph_embed(x, el, ew, W1, b1, W2, b2):
    el = el.T.reshape(2 * E)
    degp = _sc_deg(el, ew)
    A1, s1, di = _tc_l1(x, W1, b1.reshape(1, H1), degp)
    acc1 = _sc_scat(jnp.transpose(A1), el, ew, G=2, H=H1)
    A2, s2 = _tc_l2(jnp.transpose(acc1, (0, 2, 1)), s1, di, W2, b2.reshape(1, H2))
    acc2 = _sc_scat(jnp.transpose(A2), el, ew, G=4, H=H2)
    return _tc_fin(jnp.transpose(acc2, (0, 2, 1)), s2, di), di


def kernel(memb, demb, pemb, mirna_edgelist, mirna_edgeweight,
           disease_edge_list, disease_edgeweight, pcg_edge_list, pcg_edgeweight,
           mirna_pcg_pairs, disease_pcg_pairs, mirna_disease_pairs,
           Wm1, bm1, Wm2, bm2, Wd1, bd1, Wd2, bd2, Wp1, bp1, Wp2, bp2,
           W_assoc, b_assoc, W_mp, b_mp, W_dp, b_dp):
    mh, _ = _graph_embed(memb, mirna_edgelist, mirna_edgeweight, Wm1, bm1, Wm2, bm2)
    dh, _ = _graph_embed(demb, disease_edge_list, disease_edgeweight, Wd1, bd1, Wd2, bd2)
    ph, _ = _graph_embed(pemb, pcg_edge_list, pcg_edgeweight, Wp1, bp1, Wp2, bp2)

    mhT = jnp.transpose(mh)
    dhT = jnp.transpose(dh)
    phT = jnp.transpose(ph)

    pa = _sc_pair(mirna_disease_pairs.T.reshape(2 * P), mhT, dhT, W_assoc.reshape(H2))
    pm = _sc_pair(mirna_pcg_pairs.T.reshape(2 * P), mhT, phT, W_mp.reshape(H2))
    pd = _sc_pair(disease_pcg_pairs.T.reshape(2 * P), dhT, phT, W_dp.reshape(H2))

    assoc_out = _tc_sig(pa, b_assoc)
    mirna_pcg_out = _tc_sig(pm, b_mp)
    disease_pcg_out = _tc_sig(pd, b_dp)
    return (assoc_out, mirna_pcg_out, disease_pcg_out)


# speedup vs baseline: 29.5771x; 1.1264x over previous
import functools

import jax
import jax.numpy as jnp
from jax import lax
from jax.experimental import pallas as pl
from jax.experimental.pallas import tpu as pltpu
from jax.experimental.pallas import tpu_sc as plsc

N = 10000
E = 320000
P = 100000
D = 128
H1 = 64
H2 = 32

NC = 2
NS = 16
NT = NC * NS
LANES = 16

EBLK = 8000
EBLK_DEG = 2000
PBLK = 2000
BN = 1000

_SC_PARAMS = pltpu.CompilerParams(needs_layout_passes=False)


@functools.cache
def _mesh():
    return plsc.VectorSubcoreMesh(
        core_axis_name="c", subcore_axis_name="s", num_cores=NC, num_subcores=NS
    )


def _wid():
    return lax.axis_index("s") * NC + lax.axis_index("c")


def _deg_body(el_hbm, w_hbm, out_hbm, cbuf, wbuf, acc):
    wid = _wid()
    zero16 = jnp.zeros((LANES,), jnp.float32)

    @pl.loop(0, N // LANES)
    def _(i):
        acc[pl.ds(i * LANES, LANES)] = zero16

    epw = E // NT
    base = wid * epw

    @pl.loop(0, epw // EBLK_DEG)
    def _(b):
        off = base + b * EBLK_DEG
        pltpu.sync_copy(el_hbm.at[pl.ds(off, EBLK_DEG)], cbuf)
        pltpu.sync_copy(w_hbm.at[pl.ds(off, EBLK_DEG)], wbuf)

        @plsc.parallel_loop(0, EBLK_DEG // LANES, unroll=4)
        def _(c):
            sl = pl.ds(c * LANES, LANES)
            cvec = lax.shift_right_logical(cbuf[sl], 16)
            plsc.addupdate_scatter(acc, [cvec], wbuf[sl])

    pltpu.sync_copy(acc, out_hbm.at[pl.ds(wid * N, N)])


def _sc_deg(el, w):
    kern = pl.kernel(
        _deg_body,
        out_type=jax.ShapeDtypeStruct((NT * N,), jnp.float32),
        mesh=_mesh(),
        compiler_params=_SC_PARAMS,
        scratch_types=[
            pltpu.VMEM((EBLK_DEG,), jnp.int32),
            pltpu.VMEM((EBLK_DEG,), jnp.float32),
            pltpu.VMEM((N,), jnp.float32),
        ],
    )
    return kern(el, w).reshape(NT, N)


def _scat_body(G, H, aT_hbm, el_hbm, w_hbm, out_hbm,
               pbuf0, wbuf0, pbuf1, wbuf1,
               t0, t1, t2, t3, a0, a1, a2, a3,
               s0, s1):
    wid = _wid()
    g = wid % G
    fg = wid // G
    fbase = fg * 4
    tables = (t0, t1, t2, t3)
    accs = (a0, a1, a2, a3)

    for j in range(4):
        pltpu.sync_copy(aT_hbm.at[pl.ds((fbase + j) * N, N)], tables[j])

    zero16 = jnp.zeros((LANES,), jnp.float32)

    @pl.loop(0, N // LANES)
    def _(i):
        sl = pl.ds(i * LANES, LANES)
        for j in range(4):
            accs[j][sl] = zero16

    epg = E // G
    base = g * epg
    nblk = epg // EBLK
    mask16 = jnp.full((LANES,), 0xFFFF, jnp.int32)

    def start(b, pbuf, wbuf, sem):
        off = base + b * EBLK
        pltpu.async_copy(el_hbm.at[pl.ds(off, EBLK)], pbuf, sem)
        pltpu.async_copy(w_hbm.at[pl.ds(off, EBLK)], wbuf, sem)

    def wait(pbuf, wbuf, sem):
        pltpu.make_async_copy(el_hbm.at[pl.ds(0, EBLK)], pbuf, sem).wait()
        pltpu.make_async_copy(w_hbm.at[pl.ds(0, EBLK)], wbuf, sem).wait()

    def compute(pbuf, wbuf):
        @plsc.parallel_loop(0, EBLK // LANES, unroll=8)
        def _(c):
            sl = pl.ds(c * LANES, LANES)
            pk = pbuf[sl]
            rvec = pk & mask16
            cvec = lax.shift_right_logical(pk, 16)
            wv = wbuf[sl]
            for j in range(4):
                gv = plsc.load_gather(tables[j], [rvec])
                plsc.addupdate_scatter(accs[j], [cvec], gv * wv)

    start(0, pbuf0, wbuf0, s0)

    @pl.loop(0, nblk // 2)
    def _(h):
        b0 = 2 * h
        start(b0 + 1, pbuf1, wbuf1, s1)
        wait(pbuf0, wbuf0, s0)
        compute(pbuf0, wbuf0)

        @pl.when(b0 + 2 < nblk)
        def _():
            start(b0 + 2, pbuf0, wbuf0, s0)

        wait(pbuf1, wbuf1, s1)
        compute(pbuf1, wbuf1)

    for j in range(4):
        pltpu.sync_copy(accs[j], out_hbm.at[pl.ds((g * H + fbase + j) * N, N)])


def _sc_scat(aT, el, w, G, H):
    kern = pl.kernel(
        functools.partial(_scat_body, G, H),
        out_type=jax.ShapeDtypeStruct((G * H * N,), jnp.float32),
        mesh=_mesh(),
        compiler_params=_SC_PARAMS,
        scratch_types=[
            pltpu.VMEM((EBLK,), jnp.int32),
            pltpu.VMEM((EBLK,), jnp.float32),
            pltpu.VMEM((EBLK,), jnp.int32),
            pltpu.VMEM((EBLK,), jnp.float32),
        ] + [pltpu.VMEM((N,), jnp.float32) for _ in range(8)]
          + [pltpu.SemaphoreType.DMA, pltpu.SemaphoreType.DMA],
    )
    return kern(aT.reshape(H * N), el, w).reshape(G, H, N)


NPF = 2
NFG = NS
NPG = NC
PPG = P // NPG
PBLK2 = 2000
NPBLK = PPG // PBLK2


def _pair_body(pr_hbm, aT_hbm, bT_hbm, wv_hbm, out_hbm,
               ibuf0, ibuf1, ta0, ta1, tb0, tb1, ob0, ob1, wvb,
               si0, si1, so0, so1):
    fg = lax.axis_index("s")
    pg = lax.axis_index("c")
    tas = (ta0, ta1)
    tbs = (tb0, tb1)
    for j in range(NPF):
        pltpu.sync_copy(aT_hbm.at[pl.ds((NPF * fg + j) * N, N)], tas[j])
        pltpu.sync_copy(bT_hbm.at[pl.ds((NPF * fg + j) * N, N)], tbs[j])
    pltpu.sync_copy(wv_hbm, wvb)

    for j in range(NPF):
        widv = jnp.full((LANES,), j, jnp.int32) + NPF * fg
        wvec = plsc.load_gather(wvb, [widv])

        @pl.loop(0, N // LANES)
        def _(i):
            sl = pl.ds(i * LANES, LANES)
            tas[j][sl] = tas[j][sl] * wvec

    pbase = pg * PPG
    obase = fg * P + pg * PPG
    mask16 = jnp.full((LANES,), 0xFFFF, jnp.int32)

    def start_in(b, ibuf, si):
        pltpu.async_copy(pr_hbm.at[pl.ds(pbase + b * PBLK2, PBLK2)], ibuf, si)

    def wait_in(ibuf, si):
        pltpu.make_async_copy(pr_hbm.at[pl.ds(0, PBLK2)], ibuf, si).wait()

    def compute(ibuf, ob):
        @plsc.parallel_loop(0, PBLK2 // LANES, unroll=8)
        def _(c):
            sl = pl.ds(c * LANES, LANES)
            pk = ibuf[sl]
            ia = pk & mask16
            ib = lax.shift_right_logical(pk, 16)
            acc = plsc.load_gather(tas[0], [ia]) * plsc.load_gather(tbs[0], [ib])
            acc = acc + plsc.load_gather(tas[1], [ia]) * plsc.load_gather(tbs[1], [ib])
            ob[sl] = acc

    def start_out(b, ob, so):
        pltpu.async_copy(ob, out_hbm.at[pl.ds(obase + b * PBLK2, PBLK2)], so)

    def wait_out(ob, so):
        pltpu.make_async_copy(ob, out_hbm.at[pl.ds(0, PBLK2)], so).wait()

    start_in(0, ibuf0, si0)

    @pl.loop(0, (NPBLK - 1) // 2)
    def _(h):
        b0 = 2 * h
        start_in(b0 + 1, ibuf1, si1)
        wait_in(ibuf0, si0)

        @pl.when(h > 0)
        def _():
            wait_out(ob0, so0)

        compute(ibuf0, ob0)
        start_out(b0, ob0, so0)
        start_in(b0 + 2, ibuf0, si0)
        wait_in(ibuf1, si1)

        @pl.when(h > 0)
        def _():
            wait_out(ob1, so1)

        compute(ibuf1, ob1)
        start_out(b0 + 1, ob1, so1)

    wait_in(ibuf0, si0)
    wait_out(ob0, so0)
    compute(ibuf0, ob0)
    start_out(NPBLK - 1, ob0, so0)
    wait_out(ob0, so0)
    wait_out(ob1, so1)


def _sc_pair(pairs, aT, bT, wcol):
    kern = pl.kernel(
        _pair_body,
        out_type=jax.ShapeDtypeStruct((NFG * P,), jnp.float32),
        mesh=_mesh(),
        compiler_params=_SC_PARAMS,
        scratch_types=[
            pltpu.VMEM((PBLK2,), jnp.int32),
            pltpu.VMEM((PBLK2,), jnp.int32),
            pltpu.VMEM((N,), jnp.float32),
            pltpu.VMEM((N,), jnp.float32),
            pltpu.VMEM((N,), jnp.float32),
            pltpu.VMEM((N,), jnp.float32),
            pltpu.VMEM((PBLK2,), jnp.float32),
            pltpu.VMEM((PBLK2,), jnp.float32),
            pltpu.VMEM((H2,), jnp.float32),
        ] + [pltpu.SemaphoreType.DMA for _ in range(4)],
    )
    return kern(pairs, aT.reshape(H2 * N), bT.reshape(H2 * N), wcol).reshape(NFG, P)


def _tcl1_body(x_ref, w_ref, b_ref, degp_ref, a_ref, s_ref, di_ref):
    deg = jnp.sum(degp_ref[...], axis=0) + 1.0
    dis = lax.rsqrt(deg)
    inv = 1.0 / deg
    xw = jnp.dot(x_ref[...], w_ref[...], preferred_element_type=jnp.float32)
    a_ref[...] = xw * dis[:, None]
    s_ref[...] = xw * inv[:, None] + b_ref[...]
    di_ref[...] = jnp.stack([dis, inv], axis=0)


def _tc_l1(x, W1, b1, degp):
    return pl.pallas_call(
        _tcl1_body,
        out_shape=[
            jax.ShapeDtypeStruct((N, H1), jnp.float32),
            jax.ShapeDtypeStruct((N, H1), jnp.float32),
            jax.ShapeDtypeStruct((2, N), jnp.float32),
        ],
    )(x, W1, b1, degp)


def _tcl2_body(acc_ref, s1_ref, di_ref, w_ref, b_ref, a_ref, s_ref):
    accsum = jnp.sum(acc_ref[...], axis=0)
    dis = di_ref[0]
    inv = di_ref[1]
    h1 = accsum * dis[:, None] + s1_ref[...]
    xw = jnp.dot(h1, w_ref[...], preferred_element_type=jnp.float32)
    a_ref[...] = xw * dis[:, None]
    s_ref[...] = xw * inv[:, None] + b_ref[...]


def _tc_l2(acc, s1, di, W2, b2):
    return pl.pallas_call(
        _tcl2_body,
        out_shape=[
            jax.ShapeDtypeStruct((N, H2), jnp.float32),
            jax.ShapeDtypeStruct((N, H2), jnp.float32),
        ],
    )(acc, s1, di, W2, b2)


def _tcfin_body(acc_ref, s2_ref, di_ref, o_ref):
    accsum = jnp.sum(acc_ref[...], axis=0)
    dis = di_ref[0]
    o_ref[...] = jax.nn.relu(accsum * dis[:, None] + s2_ref[...])


def _tc_fin(acc, s2, di):
    return pl.pallas_call(
        _tcfin_body,
        out_shape=jax.ShapeDtypeStruct((N, H2), jnp.float32),
    )(acc, s2, di)


def _tcsig_body(p_ref, b_ref, o_ref):
    s = jnp.sum(p_ref[...], axis=0) + b_ref[0, 0]
    o_ref[...] = jax.nn.sigmoid(s)[None, :]


def _tc_sig(partials, bias):
    out = pl.pallas_call(
        _tcsig_body,
        out_shape=jax.ShapeDtypeStruct((1, P), jnp.float32),
    )(partials, bias.reshape(1, 1))
    return out.reshape(P)


def _graph_embed(x, el, ew, W1, b1, W2, b2):
    el = el[:, 0] + el[:, 1] * 65536
    degp = _sc_deg(el, ew)
    A1, s1, di = _tc_l1(x, W1, b1.reshape(1, H1), degp)
    acc1 = _sc_scat(jnp.transpose(A1), el, ew, G=2, H=H1)
    A2, s2 = _tc_l2(jnp.transpose(acc1, (0, 2, 1)), s1, di, W2, b2.reshape(1, H2))
    acc2 = _sc_scat(jnp.transpose(A2), el, ew, G=4, H=H2)
    return _tc_fin(jnp.transpose(acc2, (0, 2, 1)), s2, di), di


def kernel(memb, demb, pemb, mirna_edgelist, mirna_edgeweight,
           disease_edge_list, disease_edgeweight, pcg_edge_list, pcg_edgeweight,
           mirna_pcg_pairs, disease_pcg_pairs, mirna_disease_pairs,
           Wm1, bm1, Wm2, bm2, Wd1, bd1, Wd2, bd2, Wp1, bp1, Wp2, bp2,
           W_assoc, b_assoc, W_mp, b_mp, W_dp, b_dp):
    mh, _ = _graph_embed(memb, mirna_edgelist, mirna_edgeweight, Wm1, bm1, Wm2, bm2)
    dh, _ = _graph_embed(demb, disease_edge_list, disease_edgeweight, Wd1, bd1, Wd2, bd2)
    ph, _ = _graph_embed(pemb, pcg_edge_list, pcg_edgeweight, Wp1, bp1, Wp2, bp2)

    mhT = jnp.transpose(mh)
    dhT = jnp.transpose(dh)
    phT = jnp.transpose(ph)

    pa = _sc_pair((mirna_disease_pairs[:, 0] + mirna_disease_pairs[:, 1] * 65536), mhT, dhT, W_assoc.reshape(H2))
    pm = _sc_pair((mirna_pcg_pairs[:, 0] + mirna_pcg_pairs[:, 1] * 65536), mhT, phT, W_mp.reshape(H2))
    pd = _sc_pair((disease_pcg_pairs[:, 0] + disease_pcg_pairs[:, 1] * 65536), dhT, phT, W_dp.reshape(H2))

    assoc_out = _tc_sig(pa, b_assoc)
    mirna_pcg_out = _tc_sig(pm, b_mp)
    disease_pcg_out = _tc_sig(pd, b_dp)
    return (assoc_out, mirna_pcg_out, disease_pcg_out)


# speedup vs baseline: 30.9728x; 1.0472x over previous
import functools

import jax
import jax.numpy as jnp
from jax import lax
from jax.experimental import pallas as pl
from jax.experimental.pallas import tpu as pltpu
from jax.experimental.pallas import tpu_sc as plsc

N = 10000
E = 320000
P = 100000
D = 128
H1 = 64
H2 = 32

NC = 2
NS = 16
NT = NC * NS
LANES = 16

EBLK = 8000
EBLK_DEG = 2000
PBLK = 2000
BN = 1000

_SC_PARAMS = pltpu.CompilerParams(needs_layout_passes=False)


@functools.cache
def _mesh():
    return plsc.VectorSubcoreMesh(
        core_axis_name="c", subcore_axis_name="s", num_cores=NC, num_subcores=NS
    )


def _wid():
    return lax.axis_index("s") * NC + lax.axis_index("c")


def _deg_body(el_hbm, w_hbm, out_hbm, cbuf, wbuf, acc):
    wid = _wid()
    zero16 = jnp.zeros((LANES,), jnp.float32)

    @pl.loop(0, N // LANES)
    def _(i):
        acc[pl.ds(i * LANES, LANES)] = zero16

    epw = E // NT
    base = wid * epw

    @pl.loop(0, epw // EBLK_DEG)
    def _(b):
        off = base + b * EBLK_DEG
        pltpu.sync_copy(el_hbm.at[pl.ds(off, EBLK_DEG)], cbuf)
        pltpu.sync_copy(w_hbm.at[pl.ds(off, EBLK_DEG)], wbuf)

        @plsc.parallel_loop(0, EBLK_DEG // LANES, unroll=4)
        def _(c):
            sl = pl.ds(c * LANES, LANES)
            cvec = lax.shift_right_logical(cbuf[sl], 16)
            plsc.addupdate_scatter(acc, [cvec], wbuf[sl])

    pltpu.sync_copy(acc, out_hbm.at[pl.ds(wid * N, N)])


def _sc_deg(el, w):
    kern = pl.kernel(
        _deg_body,
        out_type=jax.ShapeDtypeStruct((NT * N,), jnp.float32),
        mesh=_mesh(),
        compiler_params=_SC_PARAMS,
        scratch_types=[
            pltpu.VMEM((EBLK_DEG,), jnp.int32),
            pltpu.VMEM((EBLK_DEG,), jnp.float32),
            pltpu.VMEM((N,), jnp.float32),
        ],
    )
    return kern(el, w).reshape(NT, N)


def _scat_body(G, H, aT_hbm, el_hbm, w_hbm, out_hbm,
               pbuf0, wbuf0, pbuf1, wbuf1,
               t0, t1, t2, t3, a0, a1, a2, a3,
               s0, s1):
    wid = _wid()
    g = wid % G
    fg = wid // G
    fbase = fg * 4
    tables = (t0, t1, t2, t3)
    accs = (a0, a1, a2, a3)

    for j in range(4):
        pltpu.async_copy(aT_hbm.at[pl.ds((fbase + j) * N, N)], tables[j], s0)

    zero16 = jnp.zeros((LANES,), jnp.float32)

    @pl.loop(0, N // LANES)
    def _(i):
        sl = pl.ds(i * LANES, LANES)
        for j in range(4):
            accs[j][sl] = zero16

    for j in range(4):
        pltpu.make_async_copy(
            aT_hbm.at[pl.ds(0, N)], tables[j], s0).wait()

    epg = E // G
    base = g * epg
    nblk = epg // EBLK
    mask16 = jnp.full((LANES,), 0xFFFF, jnp.int32)

    def start(b, pbuf, wbuf, sem):
        off = base + b * EBLK
        pltpu.async_copy(el_hbm.at[pl.ds(off, EBLK)], pbuf, sem)
        pltpu.async_copy(w_hbm.at[pl.ds(off, EBLK)], wbuf, sem)

    def wait(pbuf, wbuf, sem):
        pltpu.make_async_copy(el_hbm.at[pl.ds(0, EBLK)], pbuf, sem).wait()
        pltpu.make_async_copy(w_hbm.at[pl.ds(0, EBLK)], wbuf, sem).wait()

    def compute(pbuf, wbuf):
        @plsc.parallel_loop(0, EBLK // LANES, unroll=8)
        def _(c):
            sl = pl.ds(c * LANES, LANES)
            pk = pbuf[sl]
            rvec = pk & mask16
            cvec = lax.shift_right_logical(pk, 16)
            wv = wbuf[sl]
            for j in range(4):
                gv = plsc.load_gather(tables[j], [rvec])
                plsc.addupdate_scatter(accs[j], [cvec], gv * wv)

    start(0, pbuf0, wbuf0, s0)

    @pl.loop(0, nblk // 2)
    def _(h):
        b0 = 2 * h
        start(b0 + 1, pbuf1, wbuf1, s1)
        wait(pbuf0, wbuf0, s0)
        compute(pbuf0, wbuf0)

        @pl.when(b0 + 2 < nblk)
        def _():
            start(b0 + 2, pbuf0, wbuf0, s0)

        wait(pbuf1, wbuf1, s1)
        compute(pbuf1, wbuf1)

    for j in range(4):
        pltpu.async_copy(accs[j], out_hbm.at[pl.ds((g * H + fbase + j) * N, N)], s0)
    for j in range(4):
        pltpu.make_async_copy(accs[j], out_hbm.at[pl.ds(0, N)], s0).wait()


def _sc_scat(aT, el, w, G, H):
    kern = pl.kernel(
        functools.partial(_scat_body, G, H),
        out_type=jax.ShapeDtypeStruct((G * H * N,), jnp.float32),
        mesh=_mesh(),
        compiler_params=_SC_PARAMS,
        scratch_types=[
            pltpu.VMEM((EBLK,), jnp.int32),
            pltpu.VMEM((EBLK,), jnp.float32),
            pltpu.VMEM((EBLK,), jnp.int32),
            pltpu.VMEM((EBLK,), jnp.float32),
        ] + [pltpu.VMEM((N,), jnp.float32) for _ in range(8)]
          + [pltpu.SemaphoreType.DMA, pltpu.SemaphoreType.DMA],
    )
    return kern(aT.reshape(H * N), el, w).reshape(G, H, N)


NPF = 2
NFG = NS
NPG = NC
PPG = P // NPG
PBLK2 = 2000
NPBLK = PPG // PBLK2


def _pair_body(pr_hbm, aT_hbm, bT_hbm, wv_hbm, out_hbm,
               ibuf0, ibuf1, ta0, ta1, tb0, tb1, ob0, ob1, wvb,
               si0, si1, so0, so1):
    fg = lax.axis_index("s")
    pg = lax.axis_index("c")
    tas = (ta0, ta1)
    tbs = (tb0, tb1)
    for j in range(NPF):
        pltpu.async_copy(aT_hbm.at[pl.ds((NPF * fg + j) * N, N)], tas[j], si0)
        pltpu.async_copy(bT_hbm.at[pl.ds((NPF * fg + j) * N, N)], tbs[j], si0)
    pltpu.sync_copy(wv_hbm, wvb)
    for j in range(NPF):
        pltpu.make_async_copy(aT_hbm.at[pl.ds(0, N)], tas[j], si0).wait()
        pltpu.make_async_copy(bT_hbm.at[pl.ds(0, N)], tbs[j], si0).wait()

    for j in range(NPF):
        widv = jnp.full((LANES,), j, jnp.int32) + NPF * fg
        wvec = plsc.load_gather(wvb, [widv])

        @pl.loop(0, N // LANES)
        def _(i):
            sl = pl.ds(i * LANES, LANES)
            tas[j][sl] = tas[j][sl] * wvec

    pbase = pg * PPG
    obase = fg * P + pg * PPG
    mask16 = jnp.full((LANES,), 0xFFFF, jnp.int32)

    def start_in(b, ibuf, si):
        pltpu.async_copy(pr_hbm.at[pl.ds(pbase + b * PBLK2, PBLK2)], ibuf, si)

    def wait_in(ibuf, si):
        pltpu.make_async_copy(pr_hbm.at[pl.ds(0, PBLK2)], ibuf, si).wait()

    def compute(ibuf, ob):
        @plsc.parallel_loop(0, PBLK2 // LANES, unroll=8)
        def _(c):
            sl = pl.ds(c * LANES, LANES)
            pk = ibuf[sl]
            ia = pk & mask16
            ib = lax.shift_right_logical(pk, 16)
            acc = plsc.load_gather(tas[0], [ia]) * plsc.load_gather(tbs[0], [ib])
            acc = acc + plsc.load_gather(tas[1], [ia]) * plsc.load_gather(tbs[1], [ib])
            ob[sl] = acc

    def start_out(b, ob, so):
        pltpu.async_copy(ob, out_hbm.at[pl.ds(obase + b * PBLK2, PBLK2)], so)

    def wait_out(ob, so):
        pltpu.make_async_copy(ob, out_hbm.at[pl.ds(0, PBLK2)], so).wait()

    start_in(0, ibuf0, si0)

    @pl.loop(0, (NPBLK - 1) // 2)
    def _(h):
        b0 = 2 * h
        start_in(b0 + 1, ibuf1, si1)
        wait_in(ibuf0, si0)

        @pl.when(h > 0)
        def _():
            wait_out(ob0, so0)

        compute(ibuf0, ob0)
        start_out(b0, ob0, so0)
        start_in(b0 + 2, ibuf0, si0)
        wait_in(ibuf1, si1)

        @pl.when(h > 0)
        def _():
            wait_out(ob1, so1)

        compute(ibuf1, ob1)
        start_out(b0 + 1, ob1, so1)

    wait_in(ibuf0, si0)
    wait_out(ob0, so0)
    compute(ibuf0, ob0)
    start_out(NPBLK - 1, ob0, so0)
    wait_out(ob0, so0)
    wait_out(ob1, so1)


def _sc_pair(pairs, aT, bT, wcol):
    kern = pl.kernel(
        _pair_body,
        out_type=jax.ShapeDtypeStruct((NFG * P,), jnp.float32),
        mesh=_mesh(),
        compiler_params=_SC_PARAMS,
        scratch_types=[
            pltpu.VMEM((PBLK2,), jnp.int32),
            pltpu.VMEM((PBLK2,), jnp.int32),
            pltpu.VMEM((N,), jnp.float32),
            pltpu.VMEM((N,), jnp.float32),
            pltpu.VMEM((N,), jnp.float32),
            pltpu.VMEM((N,), jnp.float32),
            pltpu.VMEM((PBLK2,), jnp.float32),
            pltpu.VMEM((PBLK2,), jnp.float32),
            pltpu.VMEM((H2,), jnp.float32),
        ] + [pltpu.SemaphoreType.DMA for _ in range(4)],
    )
    return kern(pairs, aT.reshape(H2 * N), bT.reshape(H2 * N), wcol).reshape(NFG, P)


def _tcl1_body(x_ref, w_ref, b_ref, degp_ref, a_ref, s_ref, di_ref):
    deg = jnp.sum(degp_ref[...], axis=0) + 1.0
    dis = lax.rsqrt(deg)
    inv = 1.0 / deg
    xw = jnp.dot(x_ref[...], w_ref[...], preferred_element_type=jnp.float32)
    a_ref[...] = xw * dis[:, None]
    s_ref[...] = xw * inv[:, None] + b_ref[...]
    di_ref[...] = jnp.stack([dis, inv], axis=0)


def _tc_l1(x, W1, b1, degp):
    return pl.pallas_call(
        _tcl1_body,
        out_shape=[
            jax.ShapeDtypeStruct((N, H1), jnp.float32),
            jax.ShapeDtypeStruct((N, H1), jnp.float32),
            jax.ShapeDtypeStruct((2, N), jnp.float32),
        ],
    )(x, W1, b1, degp)


def _tcl2_body(acc_ref, s1_ref, di_ref, w_ref, b_ref, a_ref, s_ref):
    accsum = jnp.sum(acc_ref[...], axis=0)
    dis = di_ref[0]
    inv = di_ref[1]
    h1 = accsum * dis[:, None] + s1_ref[...]
    xw = jnp.dot(h1, w_ref[...], preferred_element_type=jnp.float32)
    a_ref[...] = xw * dis[:, None]
    s_ref[...] = xw * inv[:, None] + b_ref[...]


def _tc_l2(acc, s1, di, W2, b2):
    return pl.pallas_call(
        _tcl2_body,
        out_shape=[
            jax.ShapeDtypeStruct((N, H2), jnp.float32),
            jax.ShapeDtypeStruct((N, H2), jnp.float32),
        ],
    )(acc, s1, di, W2, b2)


def _tcfin_body(acc_ref, s2_ref, di_ref, o_ref):
    accsum = jnp.sum(acc_ref[...], axis=0)
    dis = di_ref[0]
    o_ref[...] = jax.nn.relu(accsum * dis[:, None] + s2_ref[...])


def _tc_fin(acc, s2, di):
    return pl.pallas_call(
        _tcfin_body,
        out_shape=jax.ShapeDtypeStruct((N, H2), jnp.float32),
    )(acc, s2, di)


def _tcsig_body(p_ref, b_ref, o_ref):
    s = jnp.sum(p_ref[...], axis=0) + b_ref[0, 0]
    o_ref[...] = jax.nn.sigmoid(s)[None, :]


def _tc_sig(partials, bias):
    out = pl.pallas_call(
        _tcsig_body,
        out_shape=jax.ShapeDtypeStruct((1, P), jnp.float32),
    )(partials, bias.reshape(1, 1))
    return out.reshape(P)


def _graph_embed(x, el, ew, W1, b1, W2, b2):
    el = el[:, 0] + el[:, 1] * 65536
    degp = _sc_deg(el, ew)
    A1, s1, di = _tc_l1(x, W1, b1.reshape(1, H1), degp)
    acc1 = _sc_scat(jnp.transpose(A1), el, ew, G=2, H=H1)
    A2, s2 = _tc_l2(jnp.transpose(acc1, (0, 2, 1)), s1, di, W2, b2.reshape(1, H2))
    acc2 = _sc_scat(jnp.transpose(A2), el, ew, G=4, H=H2)
    return _tc_fin(jnp.transpose(acc2, (0, 2, 1)), s2, di), di


def kernel(memb, demb, pemb, mirna_edgelist, mirna_edgeweight,
           disease_edge_list, disease_edgeweight, pcg_edge_list, pcg_edgeweight,
           mirna_pcg_pairs, disease_pcg_pairs, mirna_disease_pairs,
           Wm1, bm1, Wm2, bm2, Wd1, bd1, Wd2, bd2, Wp1, bp1, Wp2, bp2,
           W_assoc, b_assoc, W_mp, b_mp, W_dp, b_dp):
    mh, _ = _graph_embed(memb, mirna_edgelist, mirna_edgeweight, Wm1, bm1, Wm2, bm2)
    dh, _ = _graph_embed(demb, disease_edge_list, disease_edgeweight, Wd1, bd1, Wd2, bd2)
    ph, _ = _graph_embed(pemb, pcg_edge_list, pcg_edgeweight, Wp1, bp1, Wp2, bp2)

    mhT = jnp.transpose(mh)
    dhT = jnp.transpose(dh)
    phT = jnp.transpose(ph)

    pa = _sc_pair((mirna_disease_pairs[:, 0] + mirna_disease_pairs[:, 1] * 65536), mhT, dhT, W_assoc.reshape(H2))
    pm = _sc_pair((mirna_pcg_pairs[:, 0] + mirna_pcg_pairs[:, 1] * 65536), mhT, phT, W_mp.reshape(H2))
    pd = _sc_pair((disease_pcg_pairs[:, 0] + disease_pcg_pairs[:, 1] * 65536), dhT, phT, W_dp.reshape(H2))

    assoc_out = _tc_sig(pa, b_assoc)
    mirna_pcg_out = _tc_sig(pm, b_mp)
    disease_pcg_out = _tc_sig(pd, b_dp)
    return (assoc_out, mirna_pcg_out, disease_pcg_out)
